# Initial kernel scaffold; baseline (speedup 1.0000x reference)
import jax
import jax.numpy as jnp
from jax.experimental import pallas as pl


def kernel(xh_lig, xh_context, t, mask_lig, mask_context, params):
    raise NotImplementedError("write your pallas kernel here")


# speedup vs baseline: 144.4840x; 144.4840x over previous
import jax
import jax.numpy as jnp
from jax import lax
from jax.experimental import pallas as pl
from jax.experimental.pallas import tpu as pltpu

NL = 2048; NP = 4096; ANF = 128; CNF = 128; ND = 3; NB = 32
HN = 64; ENF = 16; EF = ENF + 1; NLAYERS = 2; SUB = 2
CR = 15.0 / NLAYERS; NORMF = 100.0; NC = 1.0
TI = 128
TJ = 128


def _silu(x):
    return x * jax.nn.sigmoid(x)


def _layer_norm(h):
    m = h.mean(-1, keepdims=True)
    v = ((h - m) ** 2).mean(-1, keepdims=True)
    return (h - m) / jnp.sqrt(v + 1e-5)


def _mlp(ps, x):
    return _silu(x @ ps[0]['w'] + ps[0]['b']) @ ps[1]['w'] + ps[1]['b']


def _pair_pass(mode, h_ref, hsrc_ref, x_ref, xsrc_ref, segs_ref, sege_ref,
               jlo_ref, jhi_ref, out_ref, w1, b1, w2, b2, eattr, self_mask,
               c3w=None):
    nrt = NL // TI
    W1h = w1[0:HN]
    W1s = w1[HN:2 * HN]
    w1r = w1[2 * HN:2 * HN + 1]
    cvec = eattr @ w1[2 * HN + 1:] + b1.reshape(1, HN)
    odim = HN if mode == 'gcl' else ND

    def row_body(i, carry):
        i0 = i * TI
        hI = h_ref[pl.ds(i0, TI), :]
        xI = x_ref[pl.ds(i0, TI), :]
        sI = segs_ref[pl.ds(i0, TI), :]
        eI = sege_ref[pl.ds(i0, TI), :]
        AI = hI @ W1h
        iglob = i0 + lax.broadcasted_iota(jnp.int32, (TI, 1), 0)

        def col_body(j, acc):
            j0 = j * TJ
            hJ = hsrc_ref[pl.ds(j0, TJ), :]
            xJ = xsrc_ref[pl.ds(j0, TJ), :]
            BJ = hJ @ W1s
            d = xI[:, None, :] - xJ[None, :, :]
            r = jnp.sum(d * d, axis=-1)
            pre = (AI[:, None, :] + BJ[None, :, :]
                   + r[..., None] * w1r + cvec)
            m = _silu(_silu(pre).reshape(TI * TJ, HN) @ w2 + b2)
            jglob = j0 + lax.broadcasted_iota(jnp.int32, (1, TJ), 1)
            am = (jglob >= sI) & (jglob < eI)
            if self_mask:
                am = am & (iglob != jglob)
            amf = am.astype(jnp.float32)
            if mode == 'gcl':
                contrib = (m.reshape(TI, TJ, HN) * amf[..., None]).sum(axis=1)
            else:
                phi = jnp.tanh(m @ c3w) * CR
                wgt = phi.reshape(TI, TJ, 1) * amf[..., None]
                cd = d / (jnp.sqrt(r + 1e-8)[..., None] + NC)
                contrib = (cd * wgt).sum(axis=1)
            return acc + contrib

        acc = lax.fori_loop(jlo_ref[i], jhi_ref[i], col_body,
                            jnp.zeros((TI, odim), jnp.float32))
        out_ref[pl.ds(i0, TI), :] = acc
        return carry

    lax.fori_loop(0, nrt, row_body, 0)


def _gcl(gp, h_ref, hsrc_ref, x_ref, xsrc_ref, segs, sege, jlo, jhi,
         agg_ref, eattr, self_mask):
    _pair_pass('gcl', h_ref, hsrc_ref, x_ref, xsrc_ref, segs, sege, jlo, jhi,
               agg_ref, gp['e1']['w'], gp['e1']['b'], gp['e2']['w'],
               gp['e2']['b'], eattr, self_mask)
    h = h_ref[...]
    agg = agg_ref[...] / NORMF
    u = _silu(h @ gp['n1']['w'][0:HN] + agg @ gp['n1']['w'][HN:2 * HN]
              + gp['n1']['b'])
    h_ref[...] = h + u @ gp['n2']['w'] + gp['n2']['b']


def _cup(cp, h_ref, hsrc_ref, x_ref, xsrc_ref, segs, sege, jlo, jhi,
         dx_ref, eattr, self_mask):
    _pair_pass('cup', h_ref, hsrc_ref, x_ref, xsrc_ref, segs, sege, jlo, jhi,
               dx_ref, cp['c1']['w'], cp['c1']['b'], cp['c2']['w'],
               cp['c2']['b'], eattr, self_mask, c3w=cp['c3w'])
    x_ref[...] = x_ref[...] + dx_ref[...] / NORMF


def _net_body(treedef, refs):
    (x_l0_ref, h_l0_ref, x_p_ref, h_p0_ref, jitter_ref, t_ref,
     sll_ref, ell_ref, slp_ref, elp_ref,
     jlo_ll_ref, jhi_ll_ref, jlo_lp_ref, jhi_lp_ref,
     *rest) = refs
    vel_ref, feat_ref = rest[-7:-5]
    h_scr, hk_scr, x_scr, agg_scr, dx_scr = rest[-5:]
    params = jax.tree_util.tree_unflatten(treedef, [r[...] for r in rest[:-7]])

    t = t_ref[0]
    x_l = x_l0_ref[...] + jitter_ref[...]
    h_l = _mlp(params['atom_enc'], _layer_norm(h_l0_ref[...]))
    h_p = _mlp(params['ctx_enc'], _layer_norm(h_p0_ref[...]))
    h_l = jnp.concatenate([h_l, jnp.full((NL, 1), t, jnp.float32)], axis=1)
    h_p = jnp.concatenate([h_p, jnp.full((NP, 1), t, jnp.float32)], axis=1)
    ea_ll = params['edge_emb'][1:2]
    ea_lp = params['edge_emb'][0:1]

    pe = params['egnn']
    h_scr[...] = h_l @ pe['emb']['w'] + pe['emb']['b']
    x_scr[...] = x_l
    for blk in pe['blocks']:
        for gp in blk['gcls']:
            _gcl(gp, h_scr, h_scr, x_scr, x_scr, sll_ref, ell_ref,
                 jlo_ll_ref, jhi_ll_ref, agg_scr, ea_ll, True)
        _cup(blk['coord'], h_scr, h_scr, x_scr, x_scr, sll_ref, ell_ref,
             jlo_ll_ref, jhi_ll_ref, dx_scr, ea_ll, True)

    pc = params['cross']
    h65 = h_scr[...] @ pe['emb_out']['w'] + pe['emb_out']['b']
    h_scr[...] = h65 @ pc['emb_q']['w'] + pc['emb_q']['b']
    hk_scr[...] = h_p @ pc['emb_kv']['w'] + pc['emb_kv']['b']
    for blk in pc['blocks']:
        for gp in blk['gcls']:
            _gcl(gp, h_scr, hk_scr, x_scr, x_p_ref, slp_ref, elp_ref,
                 jlo_lp_ref, jhi_lp_ref, agg_scr, ea_lp, False)
        _cup(blk['coord'], h_scr, hk_scr, x_scr, x_p_ref, slp_ref, elp_ref,
             jlo_lp_ref, jhi_lp_ref, dx_scr, ea_lp, False)

    h_out = h_scr[...] @ pc['emb_out']['w'][:, 0:HN] + pc['emb_out']['b'][0:HN]
    feat_ref[...] = _mlp(params['atom_dec'], h_out)
    vel_ref[...] = x_scr[...] - x_l


def kernel(xh_lig, xh_context, t, mask_lig, mask_context, params):
    jitter = 1e-4 * jax.random.normal(jax.random.key(1), (NL, ND), jnp.float32)

    ml = mask_lig.astype(jnp.int32)
    mc = mask_context.astype(jnp.int32)
    s_ll = jnp.searchsorted(ml, ml, side='left').astype(jnp.int32)
    e_ll = jnp.searchsorted(ml, ml, side='right').astype(jnp.int32)
    s_lp = jnp.searchsorted(mc, ml, side='left').astype(jnp.int32)
    e_lp = jnp.searchsorted(mc, ml, side='right').astype(jnp.int32)
    nrt = NL // TI
    jlo_ll = s_ll.reshape(nrt, TI)[:, 0] // TJ
    jhi_ll = (e_ll.reshape(nrt, TI)[:, -1] + TJ - 1) // TJ
    jlo_lp = s_lp.reshape(nrt, TI)[:, 0] // TJ
    jhi_lp = (e_lp.reshape(nrt, TI)[:, -1] + TJ - 1) // TJ

    leaves, treedef = jax.tree_util.tree_flatten(params)

    x_l0 = xh_lig[:, :ND]
    h_l0 = xh_lig[:, ND:]
    x_p = xh_context[:, :ND]
    h_p0 = xh_context[:, ND:]

    smem = pl.BlockSpec(memory_space=pltpu.SMEM)
    vmem = pl.BlockSpec(memory_space=pltpu.VMEM)
    in_specs = ([vmem] * 5 + [smem]
                + [vmem] * 4 + [smem] * 4
                + [vmem] * len(leaves))

    vel, feat = pl.pallas_call(
        lambda *refs: _net_body(treedef, refs),
        out_shape=[jax.ShapeDtypeStruct((NL, ND), jnp.float32),
                   jax.ShapeDtypeStruct((NL, ANF), jnp.float32)],
        in_specs=in_specs,
        out_specs=[vmem, vmem],
        compiler_params=pltpu.CompilerParams(
            vmem_limit_bytes=100 * 1024 * 1024),
        scratch_shapes=[pltpu.VMEM((NL, HN), jnp.float32),
                        pltpu.VMEM((NP, HN), jnp.float32),
                        pltpu.VMEM((NL, ND), jnp.float32),
                        pltpu.VMEM((NL, HN), jnp.float32),
                        pltpu.VMEM((NL, ND), jnp.float32)],
    )(x_l0, h_l0, x_p, h_p0, jitter, t,
      s_ll.reshape(NL, 1), e_ll.reshape(NL, 1),
      s_lp.reshape(NL, 1), e_lp.reshape(NL, 1),
      jlo_ll, jhi_ll, jlo_lp, jhi_lp, *leaves)

    ligand_update = jnp.concatenate([vel, feat], axis=-1)
    pocket_update = jnp.zeros_like(xh_context)
    return ligand_update, pocket_update


# speedup vs baseline: 156.5741x; 1.0837x over previous
import jax
import jax.numpy as jnp
from jax import lax
from jax.experimental import pallas as pl
from jax.experimental.pallas import tpu as pltpu

NL = 2048; NP = 4096; ANF = 128; CNF = 128; ND = 3; NB = 32
HN = 64; ENF = 16; EF = ENF + 1; NLAYERS = 2; SUB = 2
CR = 15.0 / NLAYERS; NORMF = 100.0; NC = 1.0
TI = 128
TJ = 128


def _silu(x):
    return x * jax.nn.sigmoid(x)


def _layer_norm(h):
    m = h.mean(-1, keepdims=True)
    v = ((h - m) ** 2).mean(-1, keepdims=True)
    return (h - m) / jnp.sqrt(v + 1e-5)


def _mlp(ps, x):
    return _silu(x @ ps[0]['w'] + ps[0]['b']) @ ps[1]['w'] + ps[1]['b']


def _pair_pass(mode, h_ref, hsrc_ref, x_ref, xsrc_ref, segs_ref, sege_ref,
               jlo_ref, jhi_ref, out_ref, w1, b1, w2, b2, eattr, self_mask,
               c3w=None):
    nrt = NL // TI
    bf = jnp.bfloat16
    W1h = w1[0:HN]
    W1s = w1[HN:2 * HN]
    w1r = w1[2 * HN:2 * HN + 1].astype(bf)
    cvec = (eattr @ w1[2 * HN + 1:] + b1.reshape(1, HN)).astype(bf)
    w2b = w2.astype(bf)
    b2b = b2.astype(bf)
    c3b = None if c3w is None else c3w.astype(bf)
    odim = HN if mode == 'gcl' else ND

    def row_body(i, carry):
        i0 = i * TI
        hI = h_ref[pl.ds(i0, TI), :]
        xI = x_ref[pl.ds(i0, TI), :]
        sI = segs_ref[pl.ds(i0, TI), :]
        eI = sege_ref[pl.ds(i0, TI), :]
        AI = (hI @ W1h).astype(bf)
        iglob = i0 + lax.broadcasted_iota(jnp.int32, (TI, 1), 0)

        def col_body(j, acc):
            j0 = j * TJ
            hJ = hsrc_ref[pl.ds(j0, TJ), :]
            xJ = xsrc_ref[pl.ds(j0, TJ), :]
            BJ = (hJ @ W1s).astype(bf)
            d = xI[:, None, :] - xJ[None, :, :]
            r = jnp.sum(d * d, axis=-1)
            pre = (AI[:, None, :] + BJ[None, :, :]
                   + r.astype(bf)[..., None] * w1r + cvec)
            m2 = jnp.dot(_silu(pre).reshape(TI * TJ, HN), w2b,
                         preferred_element_type=jnp.float32).astype(bf)
            m = _silu(m2 + b2b)
            jglob = j0 + lax.broadcasted_iota(jnp.int32, (1, TJ), 1)
            am = (jglob >= sI) & (jglob < eI)
            if self_mask:
                am = am & (iglob != jglob)
            if mode == 'gcl':
                mm = m.reshape(TI, TJ, HN) * am.astype(bf)[..., None]
                contrib = mm.astype(jnp.float32).sum(axis=1)
            else:
                phi = jnp.tanh(jnp.dot(m, c3b,
                                       preferred_element_type=jnp.float32)
                               ).astype(bf) * bf(CR)
                wgt = (phi.reshape(TI, TJ, 1)
                       * am.astype(bf)[..., None]).astype(jnp.float32)
                cd = d / (jnp.sqrt(r + 1e-8)[..., None] + NC)
                contrib = (cd * wgt).sum(axis=1)
            return acc + contrib

        acc = lax.fori_loop(jlo_ref[i], jhi_ref[i], col_body,
                            jnp.zeros((TI, odim), jnp.float32))
        out_ref[pl.ds(i0, TI), :] = acc
        return carry

    lax.fori_loop(0, nrt, row_body, 0)


def _gcl(gp, h_ref, hsrc_ref, x_ref, xsrc_ref, segs, sege, jlo, jhi,
         agg_ref, eattr, self_mask):
    _pair_pass('gcl', h_ref, hsrc_ref, x_ref, xsrc_ref, segs, sege, jlo, jhi,
               agg_ref, gp['e1']['w'], gp['e1']['b'], gp['e2']['w'],
               gp['e2']['b'], eattr, self_mask)
    h = h_ref[...]
    agg = agg_ref[...] / NORMF
    u = _silu(h @ gp['n1']['w'][0:HN] + agg @ gp['n1']['w'][HN:2 * HN]
              + gp['n1']['b'])
    h_ref[...] = h + u @ gp['n2']['w'] + gp['n2']['b']


def _cup(cp, h_ref, hsrc_ref, x_ref, xsrc_ref, segs, sege, jlo, jhi,
         dx_ref, eattr, self_mask):
    _pair_pass('cup', h_ref, hsrc_ref, x_ref, xsrc_ref, segs, sege, jlo, jhi,
               dx_ref, cp['c1']['w'], cp['c1']['b'], cp['c2']['w'],
               cp['c2']['b'], eattr, self_mask, c3w=cp['c3w'])
    x_ref[...] = x_ref[...] + dx_ref[...] / NORMF


def _net_body(treedef, refs):
    (x_l0_ref, h_l0_ref, x_p_ref, h_p0_ref, jitter_ref, t_ref,
     sll_ref, ell_ref, slp_ref, elp_ref,
     jlo_ll_ref, jhi_ll_ref, jlo_lp_ref, jhi_lp_ref,
     *rest) = refs
    vel_ref, feat_ref = rest[-7:-5]
    h_scr, hk_scr, x_scr, agg_scr, dx_scr = rest[-5:]
    params = jax.tree_util.tree_unflatten(treedef, [r[...] for r in rest[:-7]])

    t = t_ref[0]
    x_l = x_l0_ref[...] + jitter_ref[...]
    h_l = _mlp(params['atom_enc'], _layer_norm(h_l0_ref[...]))
    h_p = _mlp(params['ctx_enc'], _layer_norm(h_p0_ref[...]))
    h_l = jnp.concatenate([h_l, jnp.full((NL, 1), t, jnp.float32)], axis=1)
    h_p = jnp.concatenate([h_p, jnp.full((NP, 1), t, jnp.float32)], axis=1)
    ea_ll = params['edge_emb'][1:2]
    ea_lp = params['edge_emb'][0:1]

    pe = params['egnn']
    h_scr[...] = h_l @ pe['emb']['w'] + pe['emb']['b']
    x_scr[...] = x_l
    for blk in pe['blocks']:
        for gp in blk['gcls']:
            _gcl(gp, h_scr, h_scr, x_scr, x_scr, sll_ref, ell_ref,
                 jlo_ll_ref, jhi_ll_ref, agg_scr, ea_ll, True)
        _cup(blk['coord'], h_scr, h_scr, x_scr, x_scr, sll_ref, ell_ref,
             jlo_ll_ref, jhi_ll_ref, dx_scr, ea_ll, True)

    pc = params['cross']
    h65 = h_scr[...] @ pe['emb_out']['w'] + pe['emb_out']['b']
    h_scr[...] = h65 @ pc['emb_q']['w'] + pc['emb_q']['b']
    hk_scr[...] = h_p @ pc['emb_kv']['w'] + pc['emb_kv']['b']
    for blk in pc['blocks']:
        for gp in blk['gcls']:
            _gcl(gp, h_scr, hk_scr, x_scr, x_p_ref, slp_ref, elp_ref,
                 jlo_lp_ref, jhi_lp_ref, agg_scr, ea_lp, False)
        _cup(blk['coord'], h_scr, hk_scr, x_scr, x_p_ref, slp_ref, elp_ref,
             jlo_lp_ref, jhi_lp_ref, dx_scr, ea_lp, False)

    h_out = h_scr[...] @ pc['emb_out']['w'][:, 0:HN] + pc['emb_out']['b'][0:HN]
    feat_ref[...] = _mlp(params['atom_dec'], h_out)
    vel_ref[...] = x_scr[...] - x_l


def kernel(xh_lig, xh_context, t, mask_lig, mask_context, params):
    jitter = 1e-4 * jax.random.normal(jax.random.key(1), (NL, ND), jnp.float32)

    ml = mask_lig.astype(jnp.int32)
    mc = mask_context.astype(jnp.int32)
    s_ll = jnp.searchsorted(ml, ml, side='left').astype(jnp.int32)
    e_ll = jnp.searchsorted(ml, ml, side='right').astype(jnp.int32)
    s_lp = jnp.searchsorted(mc, ml, side='left').astype(jnp.int32)
    e_lp = jnp.searchsorted(mc, ml, side='right').astype(jnp.int32)
    nrt = NL // TI
    jlo_ll = s_ll.reshape(nrt, TI)[:, 0] // TJ
    jhi_ll = (e_ll.reshape(nrt, TI)[:, -1] + TJ - 1) // TJ
    jlo_lp = s_lp.reshape(nrt, TI)[:, 0] // TJ
    jhi_lp = (e_lp.reshape(nrt, TI)[:, -1] + TJ - 1) // TJ

    leaves, treedef = jax.tree_util.tree_flatten(params)

    x_l0 = xh_lig[:, :ND]
    h_l0 = xh_lig[:, ND:]
    x_p = xh_context[:, :ND]
    h_p0 = xh_context[:, ND:]

    smem = pl.BlockSpec(memory_space=pltpu.SMEM)
    vmem = pl.BlockSpec(memory_space=pltpu.VMEM)
    in_specs = ([vmem] * 5 + [smem]
                + [vmem] * 4 + [smem] * 4
                + [vmem] * len(leaves))

    vel, feat = pl.pallas_call(
        lambda *refs: _net_body(treedef, refs),
        out_shape=[jax.ShapeDtypeStruct((NL, ND), jnp.float32),
                   jax.ShapeDtypeStruct((NL, ANF), jnp.float32)],
        in_specs=in_specs,
        out_specs=[vmem, vmem],
        compiler_params=pltpu.CompilerParams(
            vmem_limit_bytes=100 * 1024 * 1024),
        scratch_shapes=[pltpu.VMEM((NL, HN), jnp.float32),
                        pltpu.VMEM((NP, HN), jnp.float32),
                        pltpu.VMEM((NL, ND), jnp.float32),
                        pltpu.VMEM((NL, HN), jnp.float32),
                        pltpu.VMEM((NL, ND), jnp.float32)],
    )(x_l0, h_l0, x_p, h_p0, jitter, t,
      s_ll.reshape(NL, 1), e_ll.reshape(NL, 1),
      s_lp.reshape(NL, 1), e_lp.reshape(NL, 1),
      jlo_ll, jhi_ll, jlo_lp, jhi_lp, *leaves)

    ligand_update = jnp.concatenate([vel, feat], axis=-1)
    pocket_update = jnp.zeros_like(xh_context)
    return ligand_update, pocket_update


# speedup vs baseline: 233.7150x; 1.4927x over previous
import jax
import jax.numpy as jnp
from jax import lax
from jax.experimental import pallas as pl
from jax.experimental.pallas import tpu as pltpu

NL = 2048; NP = 4096; ANF = 128; CNF = 128; ND = 3; NB = 32
HN = 64; ENF = 16; EF = ENF + 1; NLAYERS = 2; SUB = 2
CR = 15.0 / NLAYERS; NORMF = 100.0; NC = 1.0
TI = 64
TJ = 64


def _silu(x):
    return x * jax.nn.sigmoid(x)


def _layer_norm(h):
    m = h.mean(-1, keepdims=True)
    v = ((h - m) ** 2).mean(-1, keepdims=True)
    return (h - m) / jnp.sqrt(v + 1e-5)


def _mlp(ps, x):
    return _silu(x @ ps[0]['w'] + ps[0]['b']) @ ps[1]['w'] + ps[1]['b']


def _pair_pass(mode, h_ref, hsrc_ref, x_ref, xsrc_ref, segs_ref, sege_ref,
               jlo_ref, jhi_ref, out_ref, a_scr, b_scr,
               w1, b1, w2, b2, eattr, self_mask, c3w=None):
    nrt = NL // TI
    ns = hsrc_ref.shape[0]
    bf = jnp.bfloat16
    W1h = w1[0:HN]
    W1s = w1[HN:2 * HN]
    w1r = w1[2 * HN:2 * HN + 1].astype(bf)
    cvec = eattr @ w1[2 * HN + 1:] + b1.reshape(1, HN)
    w2b = w2.astype(bf)
    b2b = b2.astype(bf)
    c3b = None if c3w is None else c3w.astype(bf)
    odim = HN if mode == 'gcl' else ND

    a_scr[...] = (h_ref[...] @ W1h + cvec).astype(bf)
    b_scr[0:ns, :] = (hsrc_ref[...] @ W1s).astype(bf)

    def row_body(i, carry):
        i0 = i * TI
        xI = x_ref[pl.ds(i0, TI), :]
        sI = segs_ref[pl.ds(i0, TI), :]
        eI = sege_ref[pl.ds(i0, TI), :]
        AI = a_scr[pl.ds(i0, TI), :]
        iglob = i0 + lax.broadcasted_iota(jnp.int32, (TI, 1), 0)

        def col_body(j, acc):
            j0 = j * TJ
            xJ = xsrc_ref[pl.ds(j0, TJ), :]
            BJ = b_scr[pl.ds(j0, TJ), :]
            d = xI[:, None, :] - xJ[None, :, :]
            r = jnp.sum(d * d, axis=-1)
            pre = (AI[:, None, :] + BJ[None, :, :]
                   + r.astype(bf)[..., None] * w1r)
            m2 = jnp.dot(_silu(pre).reshape(TI * TJ, HN), w2b,
                         preferred_element_type=jnp.float32).astype(bf)
            m = _silu(m2 + b2b)
            jglob = j0 + lax.broadcasted_iota(jnp.int32, (1, TJ), 1)
            am = (jglob >= sI) & (jglob < eI)
            if self_mask:
                am = am & (iglob != jglob)
            if mode == 'gcl':
                mm = m.reshape(TI, TJ, HN) * am.astype(bf)[..., None]
                contrib = mm.astype(jnp.float32).sum(axis=1)
            else:
                phi = jnp.tanh(jnp.dot(m, c3b,
                                       preferred_element_type=jnp.float32)
                               ).astype(bf) * bf(CR)
                wgt = (phi.reshape(TI, TJ, 1)
                       * am.astype(bf)[..., None]).astype(jnp.float32)
                cd = d / (jnp.sqrt(r + 1e-8)[..., None] + NC)
                contrib = (cd * wgt).sum(axis=1)
            return acc + contrib

        acc = lax.fori_loop(jlo_ref[i], jhi_ref[i], col_body,
                            jnp.zeros((TI, odim), jnp.float32))
        out_ref[pl.ds(i0, TI), :] = acc
        return carry

    lax.fori_loop(0, nrt, row_body, 0)


def _gcl(gp, h_ref, hsrc_ref, x_ref, xsrc_ref, segs, sege, jlo, jhi,
         agg_ref, a_scr, b_scr, eattr, self_mask):
    _pair_pass('gcl', h_ref, hsrc_ref, x_ref, xsrc_ref, segs, sege, jlo, jhi,
               agg_ref, a_scr, b_scr, gp['e1']['w'], gp['e1']['b'],
               gp['e2']['w'], gp['e2']['b'], eattr, self_mask)
    h = h_ref[...]
    agg = agg_ref[...] / NORMF
    u = _silu(h @ gp['n1']['w'][0:HN] + agg @ gp['n1']['w'][HN:2 * HN]
              + gp['n1']['b'])
    h_ref[...] = h + u @ gp['n2']['w'] + gp['n2']['b']


def _cup(cp, h_ref, hsrc_ref, x_ref, xsrc_ref, segs, sege, jlo, jhi,
         dx_ref, a_scr, b_scr, eattr, self_mask):
    _pair_pass('cup', h_ref, hsrc_ref, x_ref, xsrc_ref, segs, sege, jlo, jhi,
               dx_ref, a_scr, b_scr, cp['c1']['w'], cp['c1']['b'],
               cp['c2']['w'], cp['c2']['b'], eattr, self_mask, c3w=cp['c3w'])
    x_ref[...] = x_ref[...] + dx_ref[...] / NORMF


def _net_body(treedef, refs):
    (x_l0_ref, h_l0_ref, x_p_ref, h_p0_ref, jitter_ref, t_ref,
     sll_ref, ell_ref, slp_ref, elp_ref,
     jlo_ll_ref, jhi_ll_ref, jlo_lp_ref, jhi_lp_ref,
     *rest) = refs
    vel_ref, feat_ref = rest[-9:-7]
    h_scr, hk_scr, x_scr, agg_scr, dx_scr, a_scr, b_scr = rest[-7:]
    params = jax.tree_util.tree_unflatten(treedef, [r[...] for r in rest[:-9]])

    t = t_ref[0]
    x_l = x_l0_ref[...] + jitter_ref[...]
    h_l = _mlp(params['atom_enc'], _layer_norm(h_l0_ref[...]))
    h_p = _mlp(params['ctx_enc'], _layer_norm(h_p0_ref[...]))
    h_l = jnp.concatenate([h_l, jnp.full((NL, 1), t, jnp.float32)], axis=1)
    h_p = jnp.concatenate([h_p, jnp.full((NP, 1), t, jnp.float32)], axis=1)
    ea_ll = params['edge_emb'][1:2]
    ea_lp = params['edge_emb'][0:1]

    pe = params['egnn']
    h_scr[...] = h_l @ pe['emb']['w'] + pe['emb']['b']
    x_scr[...] = x_l
    for blk in pe['blocks']:
        for gp in blk['gcls']:
            _gcl(gp, h_scr, h_scr, x_scr, x_scr, sll_ref, ell_ref,
                 jlo_ll_ref, jhi_ll_ref, agg_scr, a_scr, b_scr, ea_ll, True)
        _cup(blk['coord'], h_scr, h_scr, x_scr, x_scr, sll_ref, ell_ref,
             jlo_ll_ref, jhi_ll_ref, dx_scr, a_scr, b_scr, ea_ll, True)

    pc = params['cross']
    h65 = h_scr[...] @ pe['emb_out']['w'] + pe['emb_out']['b']
    h_scr[...] = h65 @ pc['emb_q']['w'] + pc['emb_q']['b']
    hk_scr[...] = h_p @ pc['emb_kv']['w'] + pc['emb_kv']['b']
    for blk in pc['blocks']:
        for gp in blk['gcls']:
            _gcl(gp, h_scr, hk_scr, x_scr, x_p_ref, slp_ref, elp_ref,
                 jlo_lp_ref, jhi_lp_ref, agg_scr, a_scr, b_scr, ea_lp, False)
        _cup(blk['coord'], h_scr, hk_scr, x_scr, x_p_ref, slp_ref, elp_ref,
             jlo_lp_ref, jhi_lp_ref, dx_scr, a_scr, b_scr, ea_lp, False)

    h_out = h_scr[...] @ pc['emb_out']['w'][:, 0:HN] + pc['emb_out']['b'][0:HN]
    feat_ref[...] = _mlp(params['atom_dec'], h_out)
    vel_ref[...] = x_scr[...] - x_l


def kernel(xh_lig, xh_context, t, mask_lig, mask_context, params):
    jitter = 1e-4 * jax.random.normal(jax.random.key(1), (NL, ND), jnp.float32)

    ml = mask_lig.astype(jnp.int32)
    mc = mask_context.astype(jnp.int32)
    s_ll = jnp.searchsorted(ml, ml, side='left').astype(jnp.int32)
    e_ll = jnp.searchsorted(ml, ml, side='right').astype(jnp.int32)
    s_lp = jnp.searchsorted(mc, ml, side='left').astype(jnp.int32)
    e_lp = jnp.searchsorted(mc, ml, side='right').astype(jnp.int32)
    nrt = NL // TI
    jlo_ll = s_ll.reshape(nrt, TI)[:, 0] // TJ
    jhi_ll = (e_ll.reshape(nrt, TI)[:, -1] + TJ - 1) // TJ
    jlo_lp = s_lp.reshape(nrt, TI)[:, 0] // TJ
    jhi_lp = (e_lp.reshape(nrt, TI)[:, -1] + TJ - 1) // TJ

    leaves, treedef = jax.tree_util.tree_flatten(params)

    x_l0 = xh_lig[:, :ND]
    h_l0 = xh_lig[:, ND:]
    x_p = xh_context[:, :ND]
    h_p0 = xh_context[:, ND:]

    smem = pl.BlockSpec(memory_space=pltpu.SMEM)
    vmem = pl.BlockSpec(memory_space=pltpu.VMEM)
    in_specs = ([vmem] * 5 + [smem]
                + [vmem] * 4 + [smem] * 4
                + [vmem] * len(leaves))

    vel, feat = pl.pallas_call(
        lambda *refs: _net_body(treedef, refs),
        out_shape=[jax.ShapeDtypeStruct((NL, ND), jnp.float32),
                   jax.ShapeDtypeStruct((NL, ANF), jnp.float32)],
        in_specs=in_specs,
        out_specs=[vmem, vmem],
        compiler_params=pltpu.CompilerParams(
            vmem_limit_bytes=100 * 1024 * 1024),
        scratch_shapes=[pltpu.VMEM((NL, HN), jnp.float32),
                        pltpu.VMEM((NP, HN), jnp.float32),
                        pltpu.VMEM((NL, ND), jnp.float32),
                        pltpu.VMEM((NL, HN), jnp.float32),
                        pltpu.VMEM((NL, ND), jnp.float32),
                        pltpu.VMEM((NL, HN), jnp.bfloat16),
                        pltpu.VMEM((NP, HN), jnp.bfloat16)],
    )(x_l0, h_l0, x_p, h_p0, jitter, t,
      s_ll.reshape(NL, 1), e_ll.reshape(NL, 1),
      s_lp.reshape(NL, 1), e_lp.reshape(NL, 1),
      jlo_ll, jhi_ll, jlo_lp, jhi_lp, *leaves)

    ligand_update = jnp.concatenate([vel, feat], axis=-1)
    pocket_update = jnp.zeros_like(xh_context)
    return ligand_update, pocket_update
